# Initial kernel scaffold; baseline (speedup 1.0000x reference)
import jax
import jax.numpy as jnp
from jax.experimental import pallas as pl


def kernel(x, W):
    raise NotImplementedError("write your pallas kernel here")


# speedup vs baseline: 1.0573x; 1.0573x over previous
import functools

import jax
import jax.numpy as jnp
from jax.experimental import pallas as pl

TOPK = 8
NEXP = 64
BLK = 512


def _gate_kernel(x_ref, w_ref, ow_ref, oi_ref):
    x = x_ref[...]
    w = w_ref[...]
    logits = jax.lax.dot_general(
        x, w,
        dimension_numbers=(((1,), (1,)), ((), ())),
        preferred_element_type=jnp.float32,
    )
    b = logits.shape[0]
    lane = jax.lax.broadcasted_iota(jnp.int32, (b, NEXP), 1)
    vals = logits
    top_vals = []
    top_idxs = []
    for _ in range(TOPK):
        m = jnp.max(vals, axis=-1, keepdims=True)
        idx = jnp.min(jnp.where(vals == m, lane, NEXP), axis=-1, keepdims=True)
        top_vals.append(m)
        top_idxs.append(idx)
        vals = jnp.where(lane == idx, -jnp.inf, vals)
    tv = jnp.concatenate(top_vals, axis=1)
    ti = jnp.concatenate(top_idxs, axis=1)
    e = jnp.exp(tv - tv[:, :1])
    ow_ref[...] = e / jnp.sum(e, axis=-1, keepdims=True)
    oi_ref[...] = ti


@functools.partial(jax.jit, static_argnames=())
def kernel(x, W):
    n, d = x.shape
    grid = (n // BLK,)
    ow, oi = pl.pallas_call(
        _gate_kernel,
        grid=grid,
        in_specs=[
            pl.BlockSpec((BLK, d), lambda i: (i, 0)),
            pl.BlockSpec((NEXP, d), lambda i: (0, 0)),
        ],
        out_specs=[
            pl.BlockSpec((BLK, TOPK), lambda i: (i, 0)),
            pl.BlockSpec((BLK, TOPK), lambda i: (i, 0)),
        ],
        out_shape=[
            jax.ShapeDtypeStruct((n, TOPK), jnp.float32),
            jax.ShapeDtypeStruct((n, TOPK), jnp.int32),
        ],
    )(x, W)
    return ow.astype(x.dtype), oi
